# Initial kernel scaffold; baseline (speedup 1.0000x reference)
import jax
import jax.numpy as jnp
from jax.experimental import pallas as pl


def kernel(x, tok_table, pos_table):
    raise NotImplementedError("write your pallas kernel here")


# speedup vs baseline: 1.0084x; 1.0084x over previous
import functools

import jax
import jax.numpy as jnp
from jax import lax
from jax.experimental import pallas as pl
from jax.experimental.pallas import tpu as pltpu
from jax.experimental.pallas import tpu_sc as plsc

VOCAB = 100000
D = 768
B = 4
T = 4096
N = B * T

_info = plsc.get_sparse_core_info()
NC, NS, L = _info.num_cores, _info.num_subcores, _info.num_lanes
NW = NC * NS
PER_W = N // NW
CH = 32
NCHUNK = PER_W // CH


def _emb_body(tok_hbm, xf_hbm, pos_hbm, out_hbm, idx_v, rows_v, pos_v, gsem, psem):
    wid = lax.axis_index("s") * NC + lax.axis_index("c")
    base = wid * PER_W
    pltpu.sync_copy(xf_hbm.at[pl.ds(base, PER_W)], idx_v)
    t0 = lax.rem(base, T)

    def chunk_body(c, carry):
        r0 = c * CH
        g = pltpu.async_copy(tok_hbm.at[idx_v.at[pl.ds(r0, CH)]], rows_v, gsem)
        p = pltpu.async_copy(pos_hbm.at[pl.ds(t0 + r0, CH)], pos_v, psem)
        g.wait()
        p.wait()

        def row_body(r, c2):
            for j in range(D // L):
                sl = pl.ds(j * L, L)
                rows_v[r, sl] = rows_v[r, sl] + pos_v[r, sl]
            return c2

        lax.fori_loop(0, CH, row_body, 0)
        pltpu.sync_copy(rows_v, out_hbm.at[pl.ds(base + r0, CH)])
        return carry

    lax.fori_loop(0, NCHUNK, chunk_body, 0)


@functools.partial(
    pl.kernel,
    mesh=plsc.VectorSubcoreMesh(core_axis_name="c", subcore_axis_name="s"),
    out_type=jax.ShapeDtypeStruct((N, D), jnp.float32),
    scratch_types=[
        pltpu.VMEM((PER_W,), jnp.int32),
        pltpu.VMEM((CH, D), jnp.float32),
        pltpu.VMEM((CH, D), jnp.float32),
        pltpu.SemaphoreType.DMA,
        pltpu.SemaphoreType.DMA,
    ],
)
def _emb_kernel(tok_hbm, xf_hbm, pos_hbm, out_hbm, idx_v, rows_v, pos_v, gsem, psem):
    _emb_body(tok_hbm, xf_hbm, pos_hbm, out_hbm, idx_v, rows_v, pos_v, gsem, psem)


def kernel(x, tok_table, pos_table):
    b, t = x.shape
    xf = x.reshape(-1).astype(jnp.int32)
    out = _emb_kernel(tok_table, xf, pos_table)
    return out.reshape(b, t, tok_table.shape[1])


# speedup vs baseline: 1.5067x; 1.4942x over previous
import functools

import jax
import jax.numpy as jnp
from jax import lax
from jax.experimental import pallas as pl
from jax.experimental.pallas import tpu as pltpu
from jax.experimental.pallas import tpu_sc as plsc

D = 768
B = 4
T = 4096
N = B * T

_info = plsc.get_sparse_core_info()
NC, NS, L = _info.num_cores, _info.num_subcores, _info.num_lanes
NW = NC * NS
PW_T = T // NW
CH = 32
NCHUNK = PW_T // CH
NSTEP = NCHUNK * B
NRING = 3


def _emb_body(tok_hbm, xf_hbm, pos_hbm, out_hbm, idx_v, rows, pos, gsem, ssem, psem):
    wid = lax.axis_index("s") * NC + lax.axis_index("c")
    t0 = wid * PW_T

    for b in range(B):
        pltpu.sync_copy(xf_hbm.at[pl.ds(b * T + t0, PW_T)], idx_v.at[b])

    def start_gather(s, k):
        c, b = s // B, s % B
        return pltpu.async_copy(
            tok_hbm.at[idx_v.at[b, pl.ds(c * CH, CH)]], rows[k], gsem[k])

    pcopy = [None] * 2
    pcopy[0] = pltpu.async_copy(pos_hbm.at[pl.ds(t0, CH)], pos[0], psem[0])
    gcopy = [None] * NRING
    scopy = [None] * NRING
    gcopy[0] = start_gather(0, 0)
    gcopy[1] = start_gather(1, 1)

    for s in range(NSTEP):
        k = s % NRING
        c, b = s // B, s % B
        q = c % 2
        gcopy[k].wait()
        if b == 0:
            pcopy[q].wait()
            if c + 1 < NCHUNK:
                pcopy[1 - q] = pltpu.async_copy(
                    pos_hbm.at[pl.ds(t0 + (c + 1) * CH, CH)], pos[1 - q], psem[1 - q])

        def row_body(r, carry, _k=k, _q=q):
            for j in range(D // L):
                sl = pl.ds(j * L, L)
                rows[_k][r, sl] = rows[_k][r, sl] + pos[_q][r, sl]
            return carry

        lax.fori_loop(0, CH, row_body, 0)

        scopy[k] = pltpu.async_copy(
            rows[k], out_hbm.at[pl.ds(b * T + t0 + c * CH, CH)], ssem[k])

        g = s + 2
        if g < NSTEP:
            kg = g % NRING
            if scopy[kg] is not None:
                scopy[kg].wait()
            gcopy[kg] = start_gather(g, kg)

    for s in (NSTEP - 2, NSTEP - 1):
        scopy[s % NRING].wait()


@functools.partial(
    pl.kernel,
    mesh=plsc.VectorSubcoreMesh(core_axis_name="c", subcore_axis_name="s"),
    out_type=jax.ShapeDtypeStruct((N, D), jnp.float32),
    scratch_types=[
        pltpu.VMEM((B, PW_T), jnp.int32),
        [pltpu.VMEM((CH, D), jnp.float32) for _ in range(NRING)],
        [pltpu.VMEM((CH, D), jnp.float32) for _ in range(2)],
        [pltpu.SemaphoreType.DMA for _ in range(NRING)],
        [pltpu.SemaphoreType.DMA for _ in range(NRING)],
        [pltpu.SemaphoreType.DMA for _ in range(2)],
    ],
)
def _emb_kernel(tok_hbm, xf_hbm, pos_hbm, out_hbm, idx_v, rows, pos, gsem, ssem, psem):
    _emb_body(tok_hbm, xf_hbm, pos_hbm, out_hbm, idx_v, rows, pos, gsem, ssem, psem)


def kernel(x, tok_table, pos_table):
    b, t = x.shape
    xf = x.reshape(-1).astype(jnp.int32)
    out = _emb_kernel(tok_table, xf, pos_table)
    return out.reshape(b, t, tok_table.shape[1])
